# baseline (speedup 1.0000x reference)
import functools

import jax
import jax.numpy as jnp
from jax import lax
from jax.experimental import pallas as pl
from jax.experimental.pallas import tpu as pltpu
from jax.experimental.pallas import tpu_sc as plsc

_H = 128
_NC = 2
_NS = 16
_CH = 128
_NB = 2


def _dense_body(x_ref, w_ref, b_ref, o_ref):
    o_ref[0] = (
        jnp.dot(x_ref[...], w_ref[...], preferred_element_type=jnp.float32)
        + b_ref[0]
    )


def _edge_transform(x, w_cat, b_cat, nblk):
    n, h = x.shape
    g = w_cat.shape[1] // h
    ni = n // nblk
    return pl.pallas_call(
        _dense_body,
        grid=(ni, g),
        in_specs=[
            pl.BlockSpec((nblk, h), lambda i, t: (i, 0)),
            pl.BlockSpec((h, h), lambda i, t: (0, t)),
            pl.BlockSpec((1, 1, h), lambda i, t: (t, 0, 0)),
        ],
        out_specs=pl.BlockSpec((1, nblk, h), lambda i, t: (t, i, 0)),
        out_shape=jax.ShapeDtypeStruct((g, n, h), jnp.float32),
    )(x, w_cat, b_cat)


def _gru_body(part_ref, xr_ref, xz_ref, xn_ref, whr_ref, whz_ref, whn_ref,
              bhn_ref, o_ref):
    p = part_ref[0] + part_ref[1]
    r = jax.nn.sigmoid(
        xr_ref[0] + jnp.dot(p, whr_ref[...], preferred_element_type=jnp.float32))
    z = jax.nn.sigmoid(
        xz_ref[0] + jnp.dot(p, whz_ref[...], preferred_element_type=jnp.float32))
    nn = jnp.tanh(
        xn_ref[0]
        + r * (jnp.dot(p, whn_ref[...], preferred_element_type=jnp.float32)
               + bhn_ref[...]))
    o_ref[...] = (1.0 - z) * nn + z * p


def _gru(part, y, whr, whz, whn, bhn, nblk):
    npad = part.shape[1]
    n = y.shape[1]
    h = y.shape[2]
    ni = n // nblk
    return pl.pallas_call(
        _gru_body,
        grid=(ni,),
        in_specs=[
            pl.BlockSpec((2, nblk, h), lambda i: (0, i, 0)),
            pl.BlockSpec((1, nblk, h), lambda i: (6, i, 0)),
            pl.BlockSpec((1, nblk, h), lambda i: (7, i, 0)),
            pl.BlockSpec((1, nblk, h), lambda i: (8, i, 0)),
            pl.BlockSpec((h, h), lambda i: (0, 0)),
            pl.BlockSpec((h, h), lambda i: (0, 0)),
            pl.BlockSpec((h, h), lambda i: (0, 0)),
            pl.BlockSpec((1, h), lambda i: (0, 0)),
        ],
        out_specs=pl.BlockSpec((nblk, h), lambda i: (i, 0)),
        out_shape=jax.ShapeDtypeStruct((n, h), jnp.float32),
    )(part, y, y, y, whr, whz, whn, bhn)


def _sc_segment_sum(ytab, src, dst, typ, zrows, n_nodes, npad, rpt):
    e = src.shape[0]
    h = ytab.shape[1]
    nw = _NC * _NS
    epw = e // nw
    nsc = 1024 // _CH
    se = nsc * _CH
    nsup = -(-epw // se)

    mesh = plsc.VectorSubcoreMesh(core_axis_name="c", subcore_axis_name="s")

    @functools.partial(
        pl.kernel,
        mesh=mesh,
        out_type=jax.ShapeDtypeStruct((_NC, npad, h), jnp.float32),
        scratch_types=[
            *[pltpu.VMEM((se,), jnp.int32) for _ in range(6)],
            *[pltpu.VMEM((nsc, _CH), jnp.int32) for _ in range(4)],
            *[pltpu.VMEM((_CH, h), jnp.float32)
              for _ in range(_NB)],
            pltpu.VMEM_SHARED((npad, h), jnp.float32),
            *[pltpu.SemaphoreType.DMA for _ in range(2 * _NB + 2)],
        ],
    )
    def sck(ytab_h, src_h, dst_h, typ_h, z_h, out_h, *rest):
        flats = rest[:6]
        g2d = rest[6:10]
        rows = rest[10:10 + _NB]
        acc = rest[10 + _NB]
        gsem = rest[11 + _NB:11 + 2 * _NB]
        ssem = rest[11 + 2 * _NB:11 + 3 * _NB]
        stsem = rest[11 + 3 * _NB:]
        sets = tuple(
            (flats[3 * p], flats[3 * p + 1], flats[3 * p + 2],
             g2d[2 * p], g2d[2 * p + 1], stsem[p])
            for p in range(2))
        cid = lax.axis_index("c")
        sid = lax.axis_index("s")
        wid = cid * _NS + sid
        base = wid * epw

        pltpu.sync_copy(z_h, acc.at[pl.ds(sid * rpt, rpt)])
        plsc.subcore_barrier()

        zero16 = jnp.zeros((16,), jnp.int32)
        junk16 = jnp.full((16,), n_nodes, jnp.int32)

        def stage(u, st):
            valid = min(se, epw - u * se)
            sfl, tfl, dfl, _, _, sem = st
            pltpu.async_copy(src_h.at[pl.ds(base + u * se, valid)],
                             sfl.at[pl.ds(0, valid)], sem)
            pltpu.async_copy(typ_h.at[pl.ds(base + u * se, valid)],
                             tfl.at[pl.ds(0, valid)], sem)
            pltpu.async_copy(dst_h.at[pl.ds(base + u * se, valid)],
                             dfl.at[pl.ds(0, valid)], sem)

        def drain_stage(u, st):
            valid = min(se, epw - u * se)
            for q in range(3):
                pltpu.make_async_copy(
                    src_h.at[pl.ds(base, valid)],
                    st[q].at[pl.ds(0, valid)], st[5]).wait()

        stage(0, sets[0])
        for u in range(nsup):
            valid = min(se, epw - u * se)
            sfl, tfl, dfl, gi2d, dj2d, _ = sets[u % 2]
            drain_stage(u, sets[u % 2])

            def cbody(j, carry):
                for k in range(_CH // 16):
                    off = j * _CH + k * 16
                    s = sfl[pl.ds(off, 16)]
                    t = tfl[pl.ds(off, 16)]
                    gi2d[j, pl.ds(k * 16, 16)] = t * n_nodes + s
                    dj2d[j, pl.ds(k * 16, 16)] = dfl[pl.ds(off, 16)]
                return carry
            lax.fori_loop(0, nsc, cbody, 0)

            for m in range(valid // 16, se // 16):
                j, k = m // (_CH // 16), m % (_CH // 16)
                gi2d[j, pl.ds(k * 16, 16)] = zero16
                dj2d[j, pl.ds(k * 16, 16)] = junk16

            for b in range(_NB):
                pltpu.async_copy(ytab_h.at[gi2d.at[b]], rows[b], gsem[b])

            if u + 1 < nsup:
                stage(u + 1, sets[(u + 1) % 2])

            def rbody(i, carry):
                c0 = _NB * i
                for b in range(_NB):
                    pltpu.make_async_copy(
                        ytab_h.at[gi2d.at[b]], rows[b], gsem[b]).wait()
                    pltpu.async_copy(rows[b], acc.at[dj2d.at[c0 + b]],
                                     ssem[b], add=True)
                for b in range(_NB):
                    pltpu.make_async_copy(
                        rows[b], acc.at[dj2d.at[b]], ssem[b]).wait()
                    pltpu.async_copy(ytab_h.at[gi2d.at[c0 + _NB + b]],
                                     rows[b], gsem[b])
                return carry
            lax.fori_loop(0, nsc // _NB - 1, rbody, 0)

            c0 = nsc - _NB
            for b in range(_NB):
                pltpu.make_async_copy(
                    ytab_h.at[gi2d.at[b]], rows[b], gsem[b]).wait()
                pltpu.async_copy(rows[b], acc.at[dj2d.at[c0 + b]],
                                 ssem[b], add=True)
            for b in range(_NB):
                pltpu.make_async_copy(
                    rows[b], acc.at[dj2d.at[b]], ssem[b]).wait()

        plsc.subcore_barrier()
        pltpu.sync_copy(acc.at[pl.ds(sid * rpt, rpt)],
                        out_h.at[cid, pl.ds(sid * rpt, rpt)])

    return sck(ytab, src, dst, typ, zrows)


def kernel(node_embeddings, source_indices, dest_indices, edge_types,
           num_edges, W_e, b_e, W_ir, b_ir, W_hr, W_iz, b_iz, W_hz, W_in,
           b_in, W_hn, b_hn):
    n, h = node_embeddings.shape
    del num_edges

    rpt = ((n + _NS - 1) // _NS + 7) // 8 * 8
    npad = rpt * _NS

    w_cat = jnp.concatenate([W_e, W_ir, W_iz, W_in], axis=1)
    b_cat = jnp.concatenate([b_e, b_ir, b_iz, b_in]).reshape(-1, 1, h)

    y = _edge_transform(node_embeddings, w_cat, b_cat, nblk=2000)
    ytab = y.reshape(-1, h)

    zrows = jnp.zeros((rpt, h), jnp.float32)
    part = _sc_segment_sum(ytab, source_indices, dest_indices, edge_types,
                           zrows, n, npad, rpt)

    return _gru(part, y, W_hr, W_hz, W_hn, b_hn.reshape(1, h), nblk=2000)


# speedup vs baseline: 1.0425x; 1.0425x over previous
import functools

import jax
import jax.numpy as jnp
from jax import lax
from jax.experimental import pallas as pl
from jax.experimental.pallas import tpu as pltpu
from jax.experimental.pallas import tpu_sc as plsc

_H = 128
_NC = 2
_NS = 16
_CH = 64
_NB = 4


def _dense_body(x_ref, w_ref, b_ref, o_ref):
    o_ref[0] = (
        jnp.dot(x_ref[...], w_ref[...], preferred_element_type=jnp.float32)
        + b_ref[0]
    )


def _edge_transform(x, w_cat, b_cat, nblk):
    n, h = x.shape
    g = w_cat.shape[1] // h
    ni = n // nblk
    return pl.pallas_call(
        _dense_body,
        grid=(ni, g),
        in_specs=[
            pl.BlockSpec((nblk, h), lambda i, t: (i, 0)),
            pl.BlockSpec((h, h), lambda i, t: (0, t)),
            pl.BlockSpec((1, 1, h), lambda i, t: (t, 0, 0)),
        ],
        out_specs=pl.BlockSpec((1, nblk, h), lambda i, t: (t, i, 0)),
        out_shape=jax.ShapeDtypeStruct((g, n, h), jnp.float32),
    )(x, w_cat, b_cat)


def _gru_body(part_ref, xr_ref, xz_ref, xn_ref, whr_ref, whz_ref, whn_ref,
              bhn_ref, o_ref):
    p = part_ref[0] + part_ref[1]
    r = jax.nn.sigmoid(
        xr_ref[0] + jnp.dot(p, whr_ref[...], preferred_element_type=jnp.float32))
    z = jax.nn.sigmoid(
        xz_ref[0] + jnp.dot(p, whz_ref[...], preferred_element_type=jnp.float32))
    nn = jnp.tanh(
        xn_ref[0]
        + r * (jnp.dot(p, whn_ref[...], preferred_element_type=jnp.float32)
               + bhn_ref[...]))
    o_ref[...] = (1.0 - z) * nn + z * p


def _gru(part, y, whr, whz, whn, bhn, nblk):
    npad = part.shape[1]
    n = y.shape[1]
    h = y.shape[2]
    ni = n // nblk
    return pl.pallas_call(
        _gru_body,
        grid=(ni,),
        in_specs=[
            pl.BlockSpec((2, nblk, h), lambda i: (0, i, 0)),
            pl.BlockSpec((1, nblk, h), lambda i: (6, i, 0)),
            pl.BlockSpec((1, nblk, h), lambda i: (7, i, 0)),
            pl.BlockSpec((1, nblk, h), lambda i: (8, i, 0)),
            pl.BlockSpec((h, h), lambda i: (0, 0)),
            pl.BlockSpec((h, h), lambda i: (0, 0)),
            pl.BlockSpec((h, h), lambda i: (0, 0)),
            pl.BlockSpec((1, h), lambda i: (0, 0)),
        ],
        out_specs=pl.BlockSpec((nblk, h), lambda i: (i, 0)),
        out_shape=jax.ShapeDtypeStruct((n, h), jnp.float32),
    )(part, y, y, y, whr, whz, whn, bhn)


def _sc_segment_sum(ytab, src, dst, typ, zrows, n_nodes, npad, rpt):
    e = src.shape[0]
    h = ytab.shape[1]
    nw = _NC * _NS
    epw = e // nw
    nsc = 1024 // _CH
    se = nsc * _CH
    nsup = -(-epw // se)

    mesh = plsc.VectorSubcoreMesh(core_axis_name="c", subcore_axis_name="s")

    @functools.partial(
        pl.kernel,
        mesh=mesh,
        out_type=jax.ShapeDtypeStruct((_NC, npad, h), jnp.float32),
        scratch_types=[
            *[pltpu.VMEM((se,), jnp.int32) for _ in range(6)],
            *[pltpu.VMEM((nsc, _CH), jnp.int32) for _ in range(4)],
            *[pltpu.VMEM((_CH, h), jnp.float32)
              for _ in range(_NB)],
            pltpu.VMEM_SHARED((npad, h), jnp.float32),
            *[pltpu.SemaphoreType.DMA for _ in range(2 * _NB + 2)],
        ],
    )
    def sck(ytab_h, src_h, dst_h, typ_h, z_h, out_h, *rest):
        flats = rest[:6]
        g2d = rest[6:10]
        rows = rest[10:10 + _NB]
        acc = rest[10 + _NB]
        gsem = rest[11 + _NB:11 + 2 * _NB]
        ssem = rest[11 + 2 * _NB:11 + 3 * _NB]
        stsem = rest[11 + 3 * _NB:]
        sets = tuple(
            (flats[3 * p], flats[3 * p + 1], flats[3 * p + 2],
             g2d[2 * p], g2d[2 * p + 1], stsem[p])
            for p in range(2))
        cid = lax.axis_index("c")
        sid = lax.axis_index("s")
        wid = cid * _NS + sid
        base = wid * epw

        pltpu.sync_copy(z_h, acc.at[pl.ds(sid * rpt, rpt)])
        plsc.subcore_barrier()

        zero16 = jnp.zeros((16,), jnp.int32)
        junk16 = jnp.full((16,), n_nodes, jnp.int32)

        def stage(u, st):
            valid = min(se, epw - u * se)
            sfl, tfl, dfl, _, _, sem = st
            pltpu.async_copy(src_h.at[pl.ds(base + u * se, valid)],
                             sfl.at[pl.ds(0, valid)], sem)
            pltpu.async_copy(typ_h.at[pl.ds(base + u * se, valid)],
                             tfl.at[pl.ds(0, valid)], sem)
            pltpu.async_copy(dst_h.at[pl.ds(base + u * se, valid)],
                             dfl.at[pl.ds(0, valid)], sem)

        def drain_stage(u, st):
            valid = min(se, epw - u * se)
            for q in range(3):
                pltpu.make_async_copy(
                    src_h.at[pl.ds(base, valid)],
                    st[q].at[pl.ds(0, valid)], st[5]).wait()

        stage(0, sets[0])
        for u in range(nsup):
            valid = min(se, epw - u * se)
            sfl, tfl, dfl, gi2d, dj2d, _ = sets[u % 2]
            drain_stage(u, sets[u % 2])

            def cbody(j, carry):
                for k in range(_CH // 16):
                    off = j * _CH + k * 16
                    s = sfl[pl.ds(off, 16)]
                    t = tfl[pl.ds(off, 16)]
                    gi2d[j, pl.ds(k * 16, 16)] = t * n_nodes + s
                    dj2d[j, pl.ds(k * 16, 16)] = dfl[pl.ds(off, 16)]
                return carry
            lax.fori_loop(0, nsc, cbody, 0)

            for m in range(valid // 16, se // 16):
                j, k = m // (_CH // 16), m % (_CH // 16)
                gi2d[j, pl.ds(k * 16, 16)] = zero16
                dj2d[j, pl.ds(k * 16, 16)] = junk16

            for b in range(_NB):
                pltpu.async_copy(ytab_h.at[gi2d.at[b]], rows[b], gsem[b])

            if u + 1 < nsup:
                stage(u + 1, sets[(u + 1) % 2])

            def rbody(i, carry):
                c0 = _NB * i
                for b in range(_NB):
                    pltpu.make_async_copy(
                        ytab_h.at[gi2d.at[b]], rows[b], gsem[b]).wait()
                    pltpu.async_copy(rows[b], acc.at[dj2d.at[c0 + b]],
                                     ssem[b], add=True)
                for b in range(_NB):
                    pltpu.make_async_copy(
                        rows[b], acc.at[dj2d.at[b]], ssem[b]).wait()
                    pltpu.async_copy(ytab_h.at[gi2d.at[c0 + _NB + b]],
                                     rows[b], gsem[b])
                return carry
            lax.fori_loop(0, nsc // _NB - 1, rbody, 0)

            c0 = nsc - _NB
            for b in range(_NB):
                pltpu.make_async_copy(
                    ytab_h.at[gi2d.at[b]], rows[b], gsem[b]).wait()
                pltpu.async_copy(rows[b], acc.at[dj2d.at[c0 + b]],
                                 ssem[b], add=True)
            for b in range(_NB):
                pltpu.make_async_copy(
                    rows[b], acc.at[dj2d.at[b]], ssem[b]).wait()

        plsc.subcore_barrier()
        pltpu.sync_copy(acc.at[pl.ds(sid * rpt, rpt)],
                        out_h.at[cid, pl.ds(sid * rpt, rpt)])

    return sck(ytab, src, dst, typ, zrows)


def kernel(node_embeddings, source_indices, dest_indices, edge_types,
           num_edges, W_e, b_e, W_ir, b_ir, W_hr, W_iz, b_iz, W_hz, W_in,
           b_in, W_hn, b_hn):
    n, h = node_embeddings.shape
    del num_edges

    rpt = ((n + _NS - 1) // _NS + 7) // 8 * 8
    npad = rpt * _NS

    w_cat = jnp.concatenate([W_e, W_ir, W_iz, W_in], axis=1)
    b_cat = jnp.concatenate([b_e, b_ir, b_iz, b_in]).reshape(-1, 1, h)

    y = _edge_transform(node_embeddings, w_cat, b_cat, nblk=2000)
    ytab = y.reshape(-1, h)

    zrows = jnp.zeros((rpt, h), jnp.float32)
    part = _sc_segment_sum(ytab, source_indices, dest_indices, edge_types,
                           zrows, n, npad, rpt)

    return _gru(part, y, W_hr, W_hz, W_hn, b_hn.reshape(1, h), nblk=2000)
